# baseline (speedup 1.0000x reference)
import jax
import jax.numpy as jnp
from jax import lax
from jax.experimental import pallas as pl
from jax.experimental.pallas import tpu as pltpu
from jax.experimental.pallas import tpu_sc as plsc

_B, _N, _D = 4, 8193, 1024
_G = 2048
_NC, _NS, _L = 2, 16, 16
_NW = _NC * _NS
_GTC = 1536
_GSC = _G - _GTC
_GPW = _GSC // _NW
_C = 2
_NGRP = _GPW // 8
_NCHUNK = _GPW // _C
_CT = _D // 128
_TCB = 32


def _body(hv_hbm, out_hbm, chbuf, rtbuf, obuf, sem_g, sem_o):
    cid = lax.axis_index("c")
    sid = lax.axis_index("s")
    wid = sid * _NC + cid
    g0w = _GTC + wid * _GPW

    def gather_descs(s, cg):
        gbase = g0w + cg * _C
        return (
            pltpu.make_async_copy(
                hv_hbm.at[pl.ds(4 * gbase + 1, 4 * _C)], chbuf[s], sem_g[s]),
            pltpu.make_async_copy(
                hv_hbm.at[pl.ds(gbase, _C)], rtbuf[s], sem_g[s]),
        )

    def issue_gather(s, cg):
        for d in gather_descs(s, cg):
            d.start()

    def wait_gather(s, cg):
        for d in gather_descs(s, cg):
            d.wait()

    def scatter_descs(gidx):
        gb8 = g0w - _GTC + gidx * 8
        return [
            pltpu.make_async_copy(
                obuf.at[pl.ds(b * 8, 8)],
                out_hbm.at[b, pl.ds(gb8, 8)], sem_o)
            for b in range(_B)
        ]

    def compute(s, cidx):
        ch = chbuf[s]
        rt = rtbuf[s]

        def iter_bc(m, carry):
            b = m // _CT
            ct = m % _CT
            row = ct * 4 + b
            cb = ct * 128
            for gl in range(_C):
                orow = b * 8 + cidx * _C + gl
                for lg4 in range(0, 8, 4):
                    loads = [[ch[4 * gl + j, row,
                                 pl.ds((lg4 + u) * _L, _L)]
                              for j in range(4)]
                             + [rt[gl, row, pl.ds((lg4 + u) * _L, _L)]]
                             for u in range(4)]
                    for u in range(4):
                        l = loads[u]
                        v = (l[0] + l[1]) + (l[2] + l[3])
                        obuf[orow, pl.ds(cb + (lg4 + u) * _L, _L)] = (
                            (v + l[4]) * 0.2)
            return carry

        lax.fori_loop(0, _B * _CT, iter_bc, 0)

    for s in range(2):
        issue_gather(s, jnp.int32(s))

    def group(gidx, carry):
        @pl.when(gidx >= 1)
        def _():
            for d in scatter_descs(gidx - 1):
                d.wait()

        for cidx in range(4):
            cg = gidx * 4 + cidx
            s = cidx % 2
            wait_gather(s, cg)
            compute(s, cidx)

            @pl.when(cg + 2 < _NCHUNK)
            def _():
                issue_gather(s, cg + 2)

        for d in scatter_descs(gidx):
            d.start()
        return carry

    lax.fori_loop(0, _NGRP, group, 0)
    for d in scatter_descs(jnp.int32(_NGRP - 1)):
        d.wait()


def _tc_body(hv_ref, out_ref, chv, rtv, sem_c, sem_r):
    t = pl.program_id(0)
    nt = pl.num_programs(0)

    def descs(par, blk):
        g0 = blk * _TCB
        return (
            pltpu.make_async_copy(
                hv_ref.at[pl.ds(4 * g0 + 1, 4 * _TCB)], chv.at[par],
                sem_c.at[par]),
            pltpu.make_async_copy(
                hv_ref.at[pl.ds(g0, _TCB)], rtv.at[par], sem_r.at[par]),
        )

    @pl.when(t == 0)
    def _():
        for par in range(2):
            for d in descs(par, par):
                d.start()

    for par in range(2):
        blk = 2 * t + par
        for d in descs(par, blk):
            d.wait()
        ch = chv[par].reshape(_TCB, 4, _B * _CT, 128)
        v = ((ch[:, 0] + ch[:, 1]) + (ch[:, 2] + ch[:, 3])) + rtv[par]
        out_ref[pl.ds(par * _TCB, _TCB)] = v * 0.2

        @pl.when(blk + 2 < 2 * nt)
        def _():
            for d in descs(par, blk + 2):
                d.start()


@jax.jit
def _run(h):
    hv = h.transpose(1, 0, 2).reshape(_N, _B, _CT, 128)
    hv = hv.transpose(0, 2, 1, 3).reshape(_N, _B * _CT, 128)

    sc_call = pl.kernel(
        _body,
        out_type=jax.ShapeDtypeStruct((_B, _GSC, _D), jnp.float32),
        mesh=plsc.VectorSubcoreMesh(
            core_axis_name="c", subcore_axis_name="s",
            num_cores=_NC, num_subcores=_NS),
        scratch_types=[
            [pltpu.VMEM((4 * _C, _B * _CT, 128), jnp.float32)
             for _ in range(2)],
            [pltpu.VMEM((_C, _B * _CT, 128), jnp.float32)
             for _ in range(2)],
            pltpu.VMEM((4 * 8, _D), jnp.float32),
            [pltpu.SemaphoreType.DMA for _ in range(2)],
            pltpu.SemaphoreType.DMA,
        ],
    )
    sc_out = sc_call(hv)

    tc_slab = pl.pallas_call(
        _tc_body,
        grid=(_GTC // (2 * _TCB),),
        in_specs=[pl.BlockSpec(memory_space=pl.ANY)],
        out_specs=pl.BlockSpec(
            (2 * _TCB, _B * _CT, 128), lambda t: (t, 0, 0)),
        out_shape=jax.ShapeDtypeStruct((_GTC, _B * _CT, 128), jnp.float32),
        scratch_shapes=[
            pltpu.VMEM((2, 4 * _TCB, _B * _CT, 128), jnp.float32),
            pltpu.VMEM((2, _TCB, _B * _CT, 128), jnp.float32),
            pltpu.SemaphoreType.DMA((2,)),
            pltpu.SemaphoreType.DMA((2,)),
        ],
    )(hv)

    tc_out = (tc_slab.reshape(_GTC, _CT, _B, 128)
              .transpose(2, 0, 1, 3).reshape(_B, _GTC, _D))
    return jnp.concatenate([tc_out, sc_out], axis=1)


def kernel(h):
    return _run(h)


# speedup vs baseline: 1.4318x; 1.4318x over previous
import jax
import jax.numpy as jnp
from jax import lax
from jax.experimental import pallas as pl
from jax.experimental.pallas import tpu as pltpu
from jax.experimental.pallas import tpu_sc as plsc

_B, _N, _D = 4, 8193, 1024
_G = 2048
_NC, _NS, _L = 2, 16, 16
_NW = _NC * _NS
_GPW = _G // _NW
_C = 2
_NGRP = _GPW // 8
_NCHUNK = _GPW // _C
_SLAB = _D // _L
_CT = _D // 128


def _body(hv_hbm, out_hbm, chbuf, rtbuf, obuf, sem_g, sem_o):
    cid = lax.axis_index("c")
    sid = lax.axis_index("s")
    wid = sid * _NC + cid
    g0w = wid * _GPW

    def gather_descs(s, cg):
        gbase = g0w + cg * _C
        return (
            pltpu.make_async_copy(
                hv_hbm.at[pl.ds(4 * gbase + 1, 4 * _C)], chbuf[s], sem_g[s]),
            pltpu.make_async_copy(
                hv_hbm.at[pl.ds(gbase, _C)], rtbuf[s], sem_g[s]),
        )

    def issue_gather(s, cg):
        for d in gather_descs(s, cg):
            d.start()

    def wait_gather(s, cg):
        for d in gather_descs(s, cg):
            d.wait()

    def scatter_descs(gidx):
        gb8 = g0w + gidx * 8
        return [
            pltpu.make_async_copy(
                obuf.at[pl.ds(b * 8, 8)],
                out_hbm.at[b, pl.ds(gb8, 8)], sem_o)
            for b in range(_B)
        ]

    def compute(s, cidx):
        ch = chbuf[s]
        rt = rtbuf[s]

        def iter_bc(m, carry):
            b = m // _CT
            ct = m % _CT
            row = ct * 4 + b
            cb = ct * 128
            for gl in range(_C):
                orow = b * 8 + cidx * _C + gl
                for lg4 in range(0, 8, 4):
                    loads = [[ch[4 * gl + j, row,
                                 pl.ds((lg4 + u) * _L, _L)]
                              for j in range(4)]
                             + [rt[gl, row, pl.ds((lg4 + u) * _L, _L)]]
                             for u in range(4)]
                    for u in range(4):
                        l = loads[u]
                        v = (l[0] + l[1]) + (l[2] + l[3])
                        obuf[orow, pl.ds(cb + (lg4 + u) * _L, _L)] = (
                            (v + l[4]) * 0.2)
            return carry

        lax.fori_loop(0, _B * _CT, iter_bc, 0)

    for s in range(2):
        issue_gather(s, jnp.int32(s))

    def group(gidx, carry):
        @pl.when(gidx >= 1)
        def _():
            for d in scatter_descs(gidx - 1):
                d.wait()

        for cidx in range(4):
            cg = gidx * 4 + cidx
            s = cidx % 2
            wait_gather(s, cg)
            compute(s, cidx)

            @pl.when(cg + 2 < _NCHUNK)
            def _():
                issue_gather(s, cg + 2)

        for d in scatter_descs(gidx):
            d.start()
        return carry

    lax.fori_loop(0, _NGRP, group, 0)
    for d in scatter_descs(jnp.int32(_NGRP - 1)):
        d.wait()


@jax.jit
def _run(h):
    hv = h.transpose(1, 0, 2).reshape(_N, _B, _CT, 128)
    hv = hv.transpose(0, 2, 1, 3).reshape(_N, _B * _CT, 128)
    call = pl.kernel(
        _body,
        out_type=jax.ShapeDtypeStruct((_B, _G, _D), jnp.float32),
        mesh=plsc.VectorSubcoreMesh(
            core_axis_name="c", subcore_axis_name="s",
            num_cores=_NC, num_subcores=_NS),
        scratch_types=[
            [pltpu.VMEM((4 * _C, _B * _CT, 128), jnp.float32)
             for _ in range(2)],
            [pltpu.VMEM((_C, _B * _CT, 128), jnp.float32)
             for _ in range(2)],
            pltpu.VMEM((4 * 8, _D), jnp.float32),
            [pltpu.SemaphoreType.DMA for _ in range(2)],
            pltpu.SemaphoreType.DMA,
        ],
    )
    return call(hv)


def kernel(h):
    return _run(h)
